# Initial kernel scaffold; baseline (speedup 1.0000x reference)
import functools

import jax
import jax.numpy as jnp
from jax import lax
from jax.experimental import pallas as pl
from jax.experimental.pallas import tpu as pltpu
from jax.experimental.pallas import tpu_sc as plsc

N = 10000
E = 320000
D = 128
EPS = 1e-5

NC = 2
NS = 16
NW = NC * NS
EPW = E // NW
K = 80
CHUNKS = EPW // K
RPS = N // NS
DW = 8


def _sc_agg_body(with_deg, *refs):
    if with_deg:
        (h_hbm, src_hbm, dst_hbm, z128_hbm, z8_hbm, ones_hbm,
         agg0_hbm, agg1_hbm, deg0_hbm, deg1_hbm,
         agg_sh, deg_sh, sidx_v, didx_v, rows_v, ones_v, sem) = refs
    else:
        (h_hbm, src_hbm, dst_hbm, z128_hbm,
         agg0_hbm, agg1_hbm,
         agg_sh, sidx_v, didx_v, rows_v, sem) = refs

    c = lax.axis_index("c")
    s = lax.axis_index("s")
    wid = s * NC + c

    rs = pl.ds(s * RPS, RPS)
    pltpu.sync_copy(z128_hbm, agg_sh.at[rs])
    if with_deg:
        pltpu.sync_copy(z8_hbm, deg_sh.at[rs])
        pltpu.sync_copy(ones_hbm, ones_v)
    plsc.subcore_barrier()

    def body(i, carry):
        base = pl.multiple_of(wid * EPW + i * K, K)
        pltpu.sync_copy(src_hbm.at[pl.ds(base, K)], sidx_v)
        pltpu.sync_copy(dst_hbm.at[pl.ds(base, K)], didx_v)
        pltpu.async_copy(h_hbm.at[sidx_v], rows_v, sem).wait()
        pltpu.sync_copy(rows_v, agg_sh.at[didx_v], add=True)
        if with_deg:
            pltpu.sync_copy(ones_v, deg_sh.at[didx_v], add=True)
        return carry

    lax.fori_loop(0, CHUNKS, body, 0)
    plsc.subcore_barrier()

    @pl.when(c == 0)
    def _():
        pltpu.sync_copy(agg_sh.at[rs], agg0_hbm.at[rs])
        if with_deg:
            pltpu.sync_copy(deg_sh.at[rs], deg0_hbm.at[rs])

    @pl.when(c == 1)
    def _():
        pltpu.sync_copy(agg_sh.at[rs], agg1_hbm.at[rs])
        if with_deg:
            pltpu.sync_copy(deg_sh.at[rs], deg1_hbm.at[rs])


def _make_sc_agg(with_deg):
    f32 = jnp.float32
    outs = [jax.ShapeDtypeStruct((N, D), f32), jax.ShapeDtypeStruct((N, D), f32)]
    scratch = [
        pltpu.VMEM_SHARED((N, D), f32),
    ]
    if with_deg:
        outs += [jax.ShapeDtypeStruct((N, DW), f32), jax.ShapeDtypeStruct((N, DW), f32)]
        scratch += [pltpu.VMEM_SHARED((N, DW), f32)]
    scratch += [
        pltpu.VMEM((K,), jnp.int32),
        pltpu.VMEM((K,), jnp.int32),
        pltpu.VMEM((K, D), f32),
    ]
    if with_deg:
        scratch += [pltpu.VMEM((K, DW), f32)]
    scratch += [pltpu.SemaphoreType.DMA]

    mesh = plsc.VectorSubcoreMesh(core_axis_name="c", subcore_axis_name="s")
    return pl.kernel(
        functools.partial(_sc_agg_body, with_deg),
        out_type=tuple(outs),
        mesh=mesh,
        scratch_types=scratch,
    )


def _dense_body(agg0_ref, agg1_ref, deg0_ref, deg1_ref, h_ref,
                wl_ref, bl_ref, wr_ref, g_ref, beta_ref, out_ref):
    agg = agg0_ref[...] + agg1_ref[...]
    deg = deg0_ref[...][:, 0:1] + deg1_ref[...][:, 0:1]
    mean = agg / jnp.maximum(deg, 1.0)
    z = (jnp.dot(mean, wl_ref[...], preferred_element_type=jnp.float32)
         + jnp.dot(h_ref[...], wr_ref[...], preferred_element_type=jnp.float32)
         + bl_ref[...][None, :])
    mu = jnp.mean(z, axis=-1, keepdims=True)
    d = z - mu
    var = jnp.mean(d * d, axis=-1, keepdims=True)
    y = d * lax.rsqrt(var + EPS) * g_ref[...][None, :] + beta_ref[...][None, :]
    out_ref[...] = jnp.maximum(y, 0.0)


_BLK = 1000


def _dense(agg0, agg1, deg0, deg1, h, W_l, b_l, W_r, g, beta):
    grid = (N // _BLK,)
    fspec = pl.BlockSpec((_BLK, D), lambda i: (i, 0))
    dspec = pl.BlockSpec((_BLK, DW), lambda i: (i, 0))
    wspec = pl.BlockSpec((D, D), lambda i: (0, 0))
    vspec = pl.BlockSpec((D,), lambda i: (0,))
    return pl.pallas_call(
        _dense_body,
        grid=grid,
        in_specs=[fspec, fspec, dspec, dspec, fspec, wspec, vspec, wspec, vspec, vspec],
        out_specs=fspec,
        out_shape=jax.ShapeDtypeStruct((N, D), jnp.float32),
    )(agg0, agg1, deg0, deg1, h, W_l, b_l, W_r, g, beta)


def kernel(x, edge_index, W0_l, b0_l, W0_r, g0, beta0, W1_l, b1_l, W1_r, g1, beta1):
    f32 = jnp.float32
    src = edge_index[0].astype(jnp.int32)
    dst = edge_index[1].astype(jnp.int32)
    z128 = jnp.zeros((RPS, D), f32)
    z8 = jnp.zeros((RPS, DW), f32)
    ones = jnp.ones((K, DW), f32)

    agg0, agg1, deg0, deg1 = _make_sc_agg(True)(x, src, dst, z128, z8, ones)
    h1 = _dense(agg0, agg1, deg0, deg1, x, W0_l, b0_l, W0_r, g0, beta0)
    agg0b, agg1b = _make_sc_agg(False)(h1, src, dst, z128)
    return _dense(agg0b, agg1b, deg0, deg1, h1, W1_l, b1_l, W1_r, g1, beta1)


# speedup vs baseline: 11.7784x; 11.7784x over previous
import functools

import jax
import jax.numpy as jnp
from jax import lax
from jax.experimental import pallas as pl
from jax.experimental.pallas import tpu as pltpu
from jax.experimental.pallas import tpu_sc as plsc

N = 10000
E = 320000
D = 128
EPS = 1e-5

NC = 2
NS = 16
NW = NC * NS
EPW = E // NW
K = 80
CHUNKS = EPW // K
NB = 3
NI = 6
RPS = 624
RPS_LAST = N - RPS * (NS - 1)
DR = 80
L = 16


def _sc_agg_body(with_deg, *refs):
    if with_deg:
        (h_hbm, eidx_hbm, z128_hbm,
         agg0_hbm, agg1_hbm, deg0_hbm, deg1_hbm,
         agg_sh, deg_sh, idx_v, rows_v, deg_v, ridx_v, isem, gsem) = refs
    else:
        (h_hbm, eidx_hbm, z128_hbm,
         agg0_hbm, agg1_hbm,
         agg_sh, idx_v, rows_v, isem, gsem) = refs

    c = lax.axis_index("c")
    s = lax.axis_index("s")
    wid = s * NC + c

    for j in range(NI):
        pltpu.async_copy(eidx_hbm.at[wid, j], idx_v.at[j], isem.at[j])
    for b in range(NB):
        pltpu.make_async_copy(
            eidx_hbm.at[wid, b], idx_v.at[b], isem.at[b]).wait()
        pltpu.async_copy(h_hbm.at[idx_v.at[b, 0]], rows_v.at[b], gsem.at[b])

    rs_a = pl.ds(s * RPS, RPS)
    rs_b = pl.ds(RPS * (NS - 1), RPS_LAST)

    @pl.when(s < NS - 1)
    def _():
        pltpu.sync_copy(z128_hbm.at[pl.ds(0, RPS)], agg_sh.at[rs_a])

    @pl.when(s == NS - 1)
    def _():
        pltpu.sync_copy(z128_hbm, agg_sh.at[rs_b])

    if with_deg:
        zv = jnp.zeros((L,), jnp.float32)

        def zbody(i, carry):
            for j in range(D // L):
                deg_v[i, pl.ds(j * L, L)] = zv
            return carry

        lax.fori_loop(0, DR, zbody, 0)
        for j in range(DR // L):
            ridx_v[pl.ds(j * L, L)] = lax.iota(jnp.int32, L) + (j * L)

        @pl.when(s == 0)
        def _():
            pltpu.sync_copy(deg_v, deg_sh)

    plsc.subcore_barrier()

    lanes = lax.iota(jnp.int32, L)

    def deg_rmw(u):
        def sbody(g, carry2):
            idxv = idx_v[u, 1, pl.ds(g * L, L)]
            for jj in range(L):
                n = idxv[jj]
                r = lax.shift_right_logical(n, 7)
                off = lax.bitwise_and(lax.shift_right_logical(n, 4), 7) * L
                lane = lax.bitwise_and(n, 15)
                v = deg_v[r, pl.ds(off, L)]
                deg_v[r, pl.ds(off, L)] = v + jnp.where(
                    lanes == lane, 1.0, 0.0)
            return carry2

        lax.fori_loop(0, K // L, sbody, 0)

    def body(g, carry):
        for u in range(NI):
            i = g * NI + u

            @pl.when(i < CHUNKS)
            def _():
                pltpu.make_async_copy(
                    h_hbm.at[idx_v.at[u, 0]], rows_v.at[u % NB],
                    gsem.at[u % NB]).wait()
                pltpu.sync_copy(
                    rows_v.at[u % NB], agg_sh.at[idx_v.at[u, 1]], add=True)

                @pl.when(i + NB < CHUNKS)
                def _():
                    pltpu.make_async_copy(
                        eidx_hbm.at[wid, i + NB], idx_v.at[(u + NB) % NI],
                        isem.at[(u + NB) % NI]).wait()
                    pltpu.async_copy(
                        h_hbm.at[idx_v.at[(u + NB) % NI, 0]],
                        rows_v.at[u % NB], gsem.at[u % NB])

                if with_deg:
                    deg_rmw(u)

                @pl.when(i + NI < CHUNKS)
                def _():
                    pltpu.async_copy(
                        eidx_hbm.at[wid, i + NI], idx_v.at[u], isem.at[u])
        return carry

    lax.fori_loop(0, (CHUNKS + NI - 1) // NI, body, 0)

    if with_deg:
        pltpu.sync_copy(deg_v, deg_sh.at[ridx_v], add=True)
    plsc.subcore_barrier()

    def dump(agg_out, deg_out):
        @pl.when(s < NS - 1)
        def _():
            pltpu.sync_copy(agg_sh.at[rs_a], agg_out.at[rs_a])

        @pl.when(s == NS - 1)
        def _():
            pltpu.sync_copy(agg_sh.at[rs_b], agg_out.at[rs_b])

        if with_deg:
            @pl.when(s == 0)
            def _():
                pltpu.sync_copy(deg_sh, deg_out)

    @pl.when(c == 0)
    def _():
        dump(agg0_hbm, deg0_hbm if with_deg else None)

    @pl.when(c == 1)
    def _():
        dump(agg1_hbm, deg1_hbm if with_deg else None)


def _make_sc_agg(with_deg):
    f32 = jnp.float32
    outs = [jax.ShapeDtypeStruct((N, D), f32), jax.ShapeDtypeStruct((N, D), f32)]
    if with_deg:
        outs += [jax.ShapeDtypeStruct((DR, D), f32), jax.ShapeDtypeStruct((DR, D), f32)]
    scratch = [pltpu.VMEM_SHARED((N, D), f32)]
    if with_deg:
        scratch += [pltpu.VMEM_SHARED((DR, D), f32)]
    scratch += [
        pltpu.VMEM((NI, 2, K), jnp.int32),
        pltpu.VMEM((NB, K, D), f32),
    ]
    if with_deg:
        scratch += [pltpu.VMEM((DR, D), f32), pltpu.VMEM((DR,), jnp.int32)]
    scratch += [pltpu.SemaphoreType.DMA((NI,)), pltpu.SemaphoreType.DMA((NB,))]

    mesh = plsc.VectorSubcoreMesh(core_axis_name="c", subcore_axis_name="s")
    return pl.kernel(
        functools.partial(_sc_agg_body, with_deg),
        out_type=tuple(outs),
        mesh=mesh,
        scratch_types=scratch,
    )


def _dense_body(agg0_ref, agg1_ref, deg0_ref, deg1_ref, h_ref,
                wl_ref, bl_ref, wr_ref, g_ref, beta_ref, out_ref):
    agg = agg0_ref[...] + agg1_ref[...]
    deg = deg0_ref[...] + deg1_ref[...]
    mean = agg / jnp.maximum(deg, 1.0)
    z = (jnp.dot(mean, wl_ref[...], preferred_element_type=jnp.float32)
         + jnp.dot(h_ref[...], wr_ref[...], preferred_element_type=jnp.float32)
         + bl_ref[...][None, :])
    mu = jnp.mean(z, axis=-1, keepdims=True)
    d = z - mu
    var = jnp.mean(d * d, axis=-1, keepdims=True)
    y = d * lax.rsqrt(var + EPS) * g_ref[...][None, :] + beta_ref[...][None, :]
    out_ref[...] = jnp.maximum(y, 0.0)


_BLK = 1000


def _dense(agg0, agg1, deg0, deg1, h, W_l, b_l, W_r, g, beta):
    grid = (N // _BLK,)
    fspec = pl.BlockSpec((_BLK, D), lambda i: (i, 0))
    dspec = pl.BlockSpec((_BLK, 1), lambda i: (i, 0))
    wspec = pl.BlockSpec((D, D), lambda i: (0, 0))
    vspec = pl.BlockSpec((D,), lambda i: (0,))
    return pl.pallas_call(
        _dense_body,
        grid=grid,
        in_specs=[fspec, fspec, dspec, dspec, fspec, wspec, vspec, wspec, vspec, vspec],
        out_specs=fspec,
        out_shape=jax.ShapeDtypeStruct((N, D), jnp.float32),
    )(agg0, agg1, deg0, deg1, h, W_l, b_l, W_r, g, beta)


def kernel(x, edge_index, W0_l, b0_l, W0_r, g0, beta0, W1_l, b1_l, W1_r, g1, beta1):
    f32 = jnp.float32
    eidx = (edge_index.astype(jnp.int32)
            .reshape(2, NW, CHUNKS, K).transpose(1, 2, 0, 3))
    z128 = jnp.zeros((RPS_LAST, D), f32)

    agg0, agg1, deg0, deg1 = _make_sc_agg(True)(x, eidx, z128)
    deg0c = deg0.reshape(-1)[:N, None]
    deg1c = deg1.reshape(-1)[:N, None]
    h1 = _dense(agg0, agg1, deg0c, deg1c, x, W0_l, b0_l, W0_r, g0, beta0)
    agg0b, agg1b = _make_sc_agg(False)(h1, eidx, z128)
    return _dense(agg0b, agg1b, deg0c, deg1c, h1, W1_l, b1_l, W1_r, g1, beta1)


# speedup vs baseline: 12.5246x; 1.0634x over previous
import functools

import jax
import jax.numpy as jnp
from jax import lax
from jax.experimental import pallas as pl
from jax.experimental.pallas import tpu as pltpu
from jax.experimental.pallas import tpu_sc as plsc

N = 10000
E = 320000
D = 128
EPS = 1e-5

NC = 2
NS = 16
NW = NC * NS
EPW = E // NW
K = 80
CHUNKS = EPW // K
NB = 3
NI = 6
RPS = 624
RPS_LAST = N - RPS * (NS - 1)
DR = 80
L = 16


def _sc_agg_body(with_deg, *refs):
    if with_deg:
        (h_hbm, src_hbm, dst_hbm, z128_hbm,
         agg0_hbm, agg1_hbm, deg0_hbm, deg1_hbm,
         agg_sh, deg_sh, idx_v, rows_v, deg_v, ridx_v,
         isem, gsem, ssem) = refs
    else:
        (h_hbm, src_hbm, dst_hbm, z128_hbm,
         agg0_hbm, agg1_hbm,
         agg_sh, idx_v, rows_v, isem, gsem, ssem) = refs

    c = lax.axis_index("c")
    s = lax.axis_index("s")
    wid = s * NC + c

    def stage_idx(j, slot):
        pltpu.async_copy(src_hbm.at[wid, j], idx_v.at[slot, 0], isem.at[slot])
        pltpu.async_copy(dst_hbm.at[wid, j], idx_v.at[slot, 1], isem.at[slot])

    def drain_idx(j, slot):
        pltpu.make_async_copy(
            src_hbm.at[wid, j], idx_v.at[slot, 0], isem.at[slot]).wait()
        pltpu.make_async_copy(
            dst_hbm.at[wid, j], idx_v.at[slot, 1], isem.at[slot]).wait()

    for j in range(NI - 1):
        stage_idx(j, j)
    for b in range(NB - 1):
        drain_idx(b, b)
        pltpu.async_copy(h_hbm.at[idx_v.at[b, 0]], rows_v.at[b], gsem.at[b])

    rs_a = pl.ds(s * RPS, RPS)
    rs_b = pl.ds(RPS * (NS - 1), RPS_LAST)

    @pl.when(s < NS - 1)
    def _():
        pltpu.sync_copy(z128_hbm.at[pl.ds(0, RPS)], agg_sh.at[rs_a])

    @pl.when(s == NS - 1)
    def _():
        pltpu.sync_copy(z128_hbm, agg_sh.at[rs_b])

    if with_deg:
        zv = jnp.zeros((L,), jnp.float32)

        def zbody(i, carry):
            for j in range(D // L):
                deg_v[i, pl.ds(j * L, L)] = zv
            return carry

        lax.fori_loop(0, DR, zbody, 0)
        for j in range(DR // L):
            ridx_v[pl.ds(j * L, L)] = lax.iota(jnp.int32, L) + (j * L)

        @pl.when(s == 0)
        def _():
            pltpu.sync_copy(deg_v, deg_sh)

    plsc.subcore_barrier()

    lanes = lax.iota(jnp.int32, L)

    def deg_rmw(u):
        def sbody(g, carry2):
            idxv = idx_v[u, 1, pl.ds(g * L, L)]
            for jj in range(L):
                n = idxv[jj]
                r = lax.shift_right_logical(n, 7)
                off = lax.bitwise_and(lax.shift_right_logical(n, 4), 7) * L
                lane = lax.bitwise_and(n, 15)
                v = deg_v[r, pl.ds(off, L)]
                deg_v[r, pl.ds(off, L)] = v + jnp.where(
                    lanes == lane, 1.0, 0.0)
            return carry2

        lax.fori_loop(0, K // L, sbody, 0)

    def scatter_desc(u):
        return pltpu.make_async_copy(
            rows_v.at[u % NB], agg_sh.at[idx_v.at[u, 1]], ssem.at[u % NB])

    def body(g, carry):
        for u in range(NI):
            i = g * NI + u

            @pl.when(i < CHUNKS)
            def _():
                @pl.when(i >= 1)
                def _():
                    scatter_desc((u - 1) % NI).wait()

                @pl.when(i + NI - 1 < CHUNKS)
                def _():
                    stage_idx(i + NI - 1, (u - 1) % NI)

                @pl.when(i + NB - 1 < CHUNKS)
                def _():
                    drain_idx(i + NB - 1, (u + NB - 1) % NI)
                    pltpu.async_copy(
                        h_hbm.at[idx_v.at[(u + NB - 1) % NI, 0]],
                        rows_v.at[(u + NB - 1) % NB], gsem.at[(u + NB - 1) % NB])

                pltpu.make_async_copy(
                    h_hbm.at[idx_v.at[u, 0]], rows_v.at[u % NB],
                    gsem.at[u % NB]).wait()
                pltpu.async_copy(
                    rows_v.at[u % NB], agg_sh.at[idx_v.at[u, 1]],
                    ssem.at[u % NB], add=True)

                if with_deg:
                    deg_rmw(u)
        return carry

    lax.fori_loop(0, (CHUNKS + NI - 1) // NI, body, 0)
    scatter_desc((CHUNKS - 1) % NI).wait()

    if with_deg:
        pltpu.sync_copy(deg_v, deg_sh.at[ridx_v], add=True)
    plsc.subcore_barrier()

    def dump(agg_out, deg_out):
        @pl.when(s < NS - 1)
        def _():
            pltpu.sync_copy(agg_sh.at[rs_a], agg_out.at[rs_a])

        @pl.when(s == NS - 1)
        def _():
            pltpu.sync_copy(agg_sh.at[rs_b], agg_out.at[rs_b])

        if with_deg:
            @pl.when(s == 0)
            def _():
                pltpu.sync_copy(deg_sh, deg_out)

    @pl.when(c == 0)
    def _():
        dump(agg0_hbm, deg0_hbm if with_deg else None)

    @pl.when(c == 1)
    def _():
        dump(agg1_hbm, deg1_hbm if with_deg else None)


def _make_sc_agg(with_deg):
    f32 = jnp.float32
    outs = [jax.ShapeDtypeStruct((N, D), f32), jax.ShapeDtypeStruct((N, D), f32)]
    if with_deg:
        outs += [jax.ShapeDtypeStruct((DR, D), f32), jax.ShapeDtypeStruct((DR, D), f32)]
    scratch = [pltpu.VMEM_SHARED((N, D), f32)]
    if with_deg:
        scratch += [pltpu.VMEM_SHARED((DR, D), f32)]
    scratch += [
        pltpu.VMEM((NI, 2, K), jnp.int32),
        pltpu.VMEM((NB, K, D), f32),
    ]
    if with_deg:
        scratch += [pltpu.VMEM((DR, D), f32), pltpu.VMEM((DR,), jnp.int32)]
    scratch += [pltpu.SemaphoreType.DMA((NI,)), pltpu.SemaphoreType.DMA((NB,)),
                pltpu.SemaphoreType.DMA((NB,))]

    mesh = plsc.VectorSubcoreMesh(core_axis_name="c", subcore_axis_name="s")
    return pl.kernel(
        functools.partial(_sc_agg_body, with_deg),
        out_type=tuple(outs),
        mesh=mesh,
        scratch_types=scratch,
    )


def _dense_body(agg0_ref, agg1_ref, deg0_ref, deg1_ref, h_ref,
                wl_ref, bl_ref, wr_ref, g_ref, beta_ref, out_ref):
    agg = agg0_ref[...] + agg1_ref[...]
    deg = deg0_ref[...] + deg1_ref[...]
    mean = agg / jnp.maximum(deg, 1.0)
    z = (jnp.dot(mean, wl_ref[...], preferred_element_type=jnp.float32)
         + jnp.dot(h_ref[...], wr_ref[...], preferred_element_type=jnp.float32)
         + bl_ref[...][None, :])
    mu = jnp.mean(z, axis=-1, keepdims=True)
    d = z - mu
    var = jnp.mean(d * d, axis=-1, keepdims=True)
    y = d * lax.rsqrt(var + EPS) * g_ref[...][None, :] + beta_ref[...][None, :]
    out_ref[...] = jnp.maximum(y, 0.0)


_BLK = 1000


def _dense(agg0, agg1, deg0, deg1, h, W_l, b_l, W_r, g, beta):
    grid = (N // _BLK,)
    fspec = pl.BlockSpec((_BLK, D), lambda i: (i, 0))
    dspec = pl.BlockSpec((_BLK, 1), lambda i: (i, 0))
    wspec = pl.BlockSpec((D, D), lambda i: (0, 0))
    vspec = pl.BlockSpec((D,), lambda i: (0,))
    return pl.pallas_call(
        _dense_body,
        grid=grid,
        in_specs=[fspec, fspec, dspec, dspec, fspec, wspec, vspec, wspec, vspec, vspec],
        out_specs=fspec,
        out_shape=jax.ShapeDtypeStruct((N, D), jnp.float32),
    )(agg0, agg1, deg0, deg1, h, W_l, b_l, W_r, g, beta)


def kernel(x, edge_index, W0_l, b0_l, W0_r, g0, beta0, W1_l, b1_l, W1_r, g1, beta1):
    f32 = jnp.float32
    src = edge_index[0].astype(jnp.int32).reshape(NW, CHUNKS, K)
    dst = edge_index[1].astype(jnp.int32).reshape(NW, CHUNKS, K)
    z128 = jnp.zeros((RPS_LAST, D), f32)

    agg0, agg1, deg0, deg1 = _make_sc_agg(True)(x, src, dst, z128)
    deg0c = deg0.reshape(-1)[:N, None]
    deg1c = deg1.reshape(-1)[:N, None]
    h1 = _dense(agg0, agg1, deg0c, deg1c, x, W0_l, b0_l, W0_r, g0, beta0)
    agg0b, agg1b = _make_sc_agg(False)(h1, src, dst, z128)
    return _dense(agg0b, agg1b, deg0c, deg1c, h1, W1_l, b1_l, W1_r, g1, beta1)


# speedup vs baseline: 12.9104x; 1.0308x over previous
import functools

import jax
import jax.numpy as jnp
from jax import lax
from jax.experimental import pallas as pl
from jax.experimental.pallas import tpu as pltpu
from jax.experimental.pallas import tpu_sc as plsc

N = 10000
E = 320000
D = 128
EPS = 1e-5

NC = 2
NS = 16
NW = NC * NS
EPW = E // NW
K = 80
CHUNKS = EPW // K
NB1 = 3
NB2 = 4

RPS = 624
RPS_LAST = N - RPS * (NS - 1)
DR = 80
L = 16


def _sc_agg_body(with_deg, *refs):
    NB = NB1 if with_deg else NB2
    NI = 2 * NB
    if with_deg:
        (h_hbm, src_hbm, dst_hbm, z128_hbm,
         agg0_hbm, agg1_hbm, deg0_hbm, deg1_hbm,
         agg_sh, deg_sh, idx_v, rows_v, deg_v, ridx_v,
         isem, gsem, ssem) = refs
    else:
        (h_hbm, src_hbm, dst_hbm, z128_hbm,
         agg0_hbm, agg1_hbm,
         agg_sh, idx_v, rows_v, isem, gsem, ssem) = refs

    c = lax.axis_index("c")
    s = lax.axis_index("s")
    wid = s * NC + c

    def stage_idx(j, slot):
        pltpu.async_copy(src_hbm.at[wid, j], idx_v.at[slot, 0], isem.at[slot])
        pltpu.async_copy(dst_hbm.at[wid, j], idx_v.at[slot, 1], isem.at[slot])

    def drain_idx(j, slot):
        pltpu.make_async_copy(
            src_hbm.at[wid, j], idx_v.at[slot, 0], isem.at[slot]).wait()
        pltpu.make_async_copy(
            dst_hbm.at[wid, j], idx_v.at[slot, 1], isem.at[slot]).wait()

    for j in range(NI - 1):
        stage_idx(j, j)
    for b in range(NB - 1):
        drain_idx(b, b)
        pltpu.async_copy(h_hbm.at[idx_v.at[b, 0]], rows_v.at[b], gsem.at[b])

    rs_a = pl.ds(s * RPS, RPS)
    rs_b = pl.ds(RPS * (NS - 1), RPS_LAST)

    @pl.when(s < NS - 1)
    def _():
        pltpu.sync_copy(z128_hbm.at[pl.ds(0, RPS)], agg_sh.at[rs_a])

    @pl.when(s == NS - 1)
    def _():
        pltpu.sync_copy(z128_hbm, agg_sh.at[rs_b])

    if with_deg:
        zv = jnp.zeros((L,), jnp.float32)

        def zbody(i, carry):
            for j in range(D // L):
                deg_v[i, pl.ds(j * L, L)] = zv
            return carry

        lax.fori_loop(0, DR, zbody, 0)
        for j in range(DR // L):
            ridx_v[pl.ds(j * L, L)] = lax.iota(jnp.int32, L) + (j * L)

        @pl.when(s == 0)
        def _():
            pltpu.sync_copy(deg_v, deg_sh)

    plsc.subcore_barrier()

    lanes = lax.iota(jnp.int32, L)

    def deg_rmw(u):
        def sbody(g, carry2):
            idxv = idx_v[u, 1, pl.ds(g * L, L)]
            for jj in range(L):
                n = idxv[jj]
                r = lax.shift_right_logical(n, 7)
                off = lax.bitwise_and(lax.shift_right_logical(n, 4), 7) * L
                lane = lax.bitwise_and(n, 15)
                v = deg_v[r, pl.ds(off, L)]
                deg_v[r, pl.ds(off, L)] = v + jnp.where(
                    lanes == lane, 1.0, 0.0)
            return carry2

        lax.fori_loop(0, K // L, sbody, 0)

    def scatter_desc(u):
        return pltpu.make_async_copy(
            rows_v.at[u % NB], agg_sh.at[idx_v.at[u, 1]], ssem.at[u % NB])

    def body(g, carry):
        for u in range(NI):
            i = g * NI + u

            @pl.when(i < CHUNKS)
            def _():
                @pl.when(i >= 1)
                def _():
                    scatter_desc((u - 1) % NI).wait()

                @pl.when(i + NI - 1 < CHUNKS)
                def _():
                    stage_idx(i + NI - 1, (u - 1) % NI)

                @pl.when(i + NB - 1 < CHUNKS)
                def _():
                    drain_idx(i + NB - 1, (u + NB - 1) % NI)
                    pltpu.async_copy(
                        h_hbm.at[idx_v.at[(u + NB - 1) % NI, 0]],
                        rows_v.at[(u + NB - 1) % NB], gsem.at[(u + NB - 1) % NB])

                pltpu.make_async_copy(
                    h_hbm.at[idx_v.at[u, 0]], rows_v.at[u % NB],
                    gsem.at[u % NB]).wait()
                pltpu.async_copy(
                    rows_v.at[u % NB], agg_sh.at[idx_v.at[u, 1]],
                    ssem.at[u % NB], add=True)

                if with_deg:
                    deg_rmw(u)
        return carry

    lax.fori_loop(0, (CHUNKS + NI - 1) // NI, body, 0)
    scatter_desc((CHUNKS - 1) % NI).wait()

    if with_deg:
        pltpu.sync_copy(deg_v, deg_sh.at[ridx_v], add=True)
    plsc.subcore_barrier()

    def dump(agg_out, deg_out):
        @pl.when(s < NS - 1)
        def _():
            pltpu.sync_copy(agg_sh.at[rs_a], agg_out.at[rs_a])

        @pl.when(s == NS - 1)
        def _():
            pltpu.sync_copy(agg_sh.at[rs_b], agg_out.at[rs_b])

        if with_deg:
            @pl.when(s == 0)
            def _():
                pltpu.sync_copy(deg_sh, deg_out)

    @pl.when(c == 0)
    def _():
        dump(agg0_hbm, deg0_hbm if with_deg else None)

    @pl.when(c == 1)
    def _():
        dump(agg1_hbm, deg1_hbm if with_deg else None)


def _make_sc_agg(with_deg):
    f32 = jnp.float32
    outs = [jax.ShapeDtypeStruct((N, D), f32), jax.ShapeDtypeStruct((N, D), f32)]
    if with_deg:
        outs += [jax.ShapeDtypeStruct((DR, D), f32), jax.ShapeDtypeStruct((DR, D), f32)]
    scratch = [pltpu.VMEM_SHARED((N, D), f32)]
    if with_deg:
        scratch += [pltpu.VMEM_SHARED((DR, D), f32)]
    NB = NB1 if with_deg else NB2
    NI = 2 * NB
    scratch += [
        pltpu.VMEM((NI, 2, K), jnp.int32),
        pltpu.VMEM((NB, K, D), f32),
    ]
    if with_deg:
        scratch += [pltpu.VMEM((DR, D), f32), pltpu.VMEM((DR,), jnp.int32)]
    scratch += [pltpu.SemaphoreType.DMA((NI,)), pltpu.SemaphoreType.DMA((NB,)),
                pltpu.SemaphoreType.DMA((NB,))]

    mesh = plsc.VectorSubcoreMesh(core_axis_name="c", subcore_axis_name="s")
    return pl.kernel(
        functools.partial(_sc_agg_body, with_deg),
        out_type=tuple(outs),
        mesh=mesh,
        scratch_types=scratch,
    )


def _dense_body(agg0_ref, agg1_ref, deg0_ref, deg1_ref, h_ref,
                wl_ref, bl_ref, wr_ref, g_ref, beta_ref, out_ref):
    agg = agg0_ref[...] + agg1_ref[...]
    deg = deg0_ref[...] + deg1_ref[...]
    mean = agg / jnp.maximum(deg, 1.0)
    z = (jnp.dot(mean, wl_ref[...], preferred_element_type=jnp.float32)
         + jnp.dot(h_ref[...], wr_ref[...], preferred_element_type=jnp.float32)
         + bl_ref[...][None, :])
    mu = jnp.mean(z, axis=-1, keepdims=True)
    d = z - mu
    var = jnp.mean(d * d, axis=-1, keepdims=True)
    y = d * lax.rsqrt(var + EPS) * g_ref[...][None, :] + beta_ref[...][None, :]
    out_ref[...] = jnp.maximum(y, 0.0)


_BLK = 1000


def _dense(agg0, agg1, deg0, deg1, h, W_l, b_l, W_r, g, beta):
    grid = (N // _BLK,)
    fspec = pl.BlockSpec((_BLK, D), lambda i: (i, 0))
    dspec = pl.BlockSpec((_BLK, 1), lambda i: (i, 0))
    wspec = pl.BlockSpec((D, D), lambda i: (0, 0))
    vspec = pl.BlockSpec((D,), lambda i: (0,))
    return pl.pallas_call(
        _dense_body,
        grid=grid,
        in_specs=[fspec, fspec, dspec, dspec, fspec, wspec, vspec, wspec, vspec, vspec],
        out_specs=fspec,
        out_shape=jax.ShapeDtypeStruct((N, D), jnp.float32),
    )(agg0, agg1, deg0, deg1, h, W_l, b_l, W_r, g, beta)


def kernel(x, edge_index, W0_l, b0_l, W0_r, g0, beta0, W1_l, b1_l, W1_r, g1, beta1):
    f32 = jnp.float32
    src = edge_index[0].astype(jnp.int32).reshape(NW, CHUNKS, K)
    dst = edge_index[1].astype(jnp.int32).reshape(NW, CHUNKS, K)
    z128 = jnp.zeros((RPS_LAST, D), f32)

    agg0, agg1, deg0, deg1 = _make_sc_agg(True)(x, src, dst, z128)
    deg0c = deg0.reshape(DR * D, 1)
    deg1c = deg1.reshape(DR * D, 1)
    h1 = _dense(agg0, agg1, deg0c, deg1c, x, W0_l, b0_l, W0_r, g0, beta0)
    agg0b, agg1b = _make_sc_agg(False)(h1, src, dst, z128)
    return _dense(agg0b, agg1b, deg0c, deg1c, h1, W1_l, b1_l, W1_r, g1, beta1)


# speedup vs baseline: 13.3742x; 1.0359x over previous
import functools

import jax
import jax.numpy as jnp
from jax import lax
from jax.experimental import pallas as pl
from jax.experimental.pallas import tpu as pltpu
from jax.experimental.pallas import tpu_sc as plsc

N = 10000
E = 320000
D = 128
EPS = 1e-5

NC = 2
NS = 16
NW = NC * NS
EPW = E // NW
K = 80
CHUNKS = EPW // K
NB1 = 3
NB2 = 4

RPS = 624
RPS_LAST = N - RPS * (NS - 1)
DR = 80
L = 16


def _sc_agg_body(with_deg, *refs):
    NB = NB1 if with_deg else NB2
    NI = 2 * NB
    if with_deg:
        (h_hbm, src_hbm, dst_hbm, z128_hbm,
         agg0_hbm, agg1_hbm, deg0_hbm, deg1_hbm,
         agg_sh, deg_sh, idx_v, rows_v, deg_v, ridx_v,
         isem, gsem, ssem) = refs
    else:
        (h_hbm, src_hbm, dst_hbm, z128_hbm,
         agg0_hbm, agg1_hbm,
         agg_sh, idx_v, rows_v, isem, gsem, ssem) = refs

    c = lax.axis_index("c")
    s = lax.axis_index("s")
    wid = s * NC + c

    def stage_idx(j, slot):
        pltpu.async_copy(src_hbm.at[wid, j], idx_v.at[slot, 0], isem.at[slot])
        pltpu.async_copy(dst_hbm.at[wid, j], idx_v.at[slot, 1], isem.at[slot])

    def drain_idx(j, slot):
        pltpu.make_async_copy(
            src_hbm.at[wid, j], idx_v.at[slot, 0], isem.at[slot]).wait()
        pltpu.make_async_copy(
            dst_hbm.at[wid, j], idx_v.at[slot, 1], isem.at[slot]).wait()

    for j in range(NI - 1):
        stage_idx(j, j)
    for b in range(NB - 1):
        drain_idx(b, b)
        pltpu.async_copy(h_hbm.at[idx_v.at[b, 0]], rows_v.at[b], gsem.at[b])

    rs_a = pl.ds(s * RPS, RPS)
    rs_b = pl.ds(RPS * (NS - 1), RPS_LAST)

    @pl.when(s < NS - 1)
    def _():
        pltpu.sync_copy(z128_hbm.at[pl.ds(0, RPS)], agg_sh.at[rs_a])

    @pl.when(s == NS - 1)
    def _():
        pltpu.sync_copy(z128_hbm, agg_sh.at[rs_b])

    if with_deg:
        zv = jnp.zeros((L,), jnp.float32)

        def zbody(i, carry):
            for j in range(D // L):
                deg_v[i, pl.ds(j * L, L)] = zv
            return carry

        lax.fori_loop(0, DR, zbody, 0)
        for j in range(DR // L):
            ridx_v[pl.ds(j * L, L)] = lax.iota(jnp.int32, L) + (j * L)

        @pl.when(s == 0)
        def _():
            pltpu.sync_copy(deg_v, deg_sh)

    plsc.subcore_barrier()

    lanes = lax.iota(jnp.int32, L)

    def deg_rmw(u):
        def sbody(g, carry2):
            idxv = idx_v[u, 1, pl.ds(g * L, L)]
            for jj in range(L):
                n = idxv[jj]
                r = lax.shift_right_logical(n, 7)
                off = lax.bitwise_and(lax.shift_right_logical(n, 4), 7) * L
                lane = lax.bitwise_and(n, 15)
                v = deg_v[r, pl.ds(off, L)]
                deg_v[r, pl.ds(off, L)] = v + jnp.where(
                    lanes == lane, 1.0, 0.0)
            return carry2

        lax.fori_loop(0, K // L, sbody, 0)

    def scatter_desc(u):
        return pltpu.make_async_copy(
            rows_v.at[u % NB], agg_sh.at[idx_v.at[u, 1]], ssem.at[u % NB])

    def body(g, carry):
        for u in range(NI):
            i = g * NI + u

            @pl.when(i < CHUNKS)
            def _():
                @pl.when(i >= 1)
                def _():
                    scatter_desc((u - 1) % NI).wait()

                @pl.when(i + NI - 1 < CHUNKS)
                def _():
                    stage_idx(i + NI - 1, (u - 1) % NI)

                @pl.when(i + NB - 1 < CHUNKS)
                def _():
                    drain_idx(i + NB - 1, (u + NB - 1) % NI)
                    pltpu.async_copy(
                        h_hbm.at[idx_v.at[(u + NB - 1) % NI, 0]],
                        rows_v.at[(u + NB - 1) % NB], gsem.at[(u + NB - 1) % NB])

                pltpu.make_async_copy(
                    h_hbm.at[idx_v.at[u, 0]], rows_v.at[u % NB],
                    gsem.at[u % NB]).wait()
                pltpu.async_copy(
                    rows_v.at[u % NB], agg_sh.at[idx_v.at[u, 1]],
                    ssem.at[u % NB], add=True)

                if with_deg:
                    deg_rmw(u)
        return carry

    lax.fori_loop(0, (CHUNKS + NI - 1) // NI, body, 0)
    scatter_desc((CHUNKS - 1) % NI).wait()

    if with_deg:
        pltpu.sync_copy(deg_v, deg_sh.at[ridx_v], add=True)
    plsc.subcore_barrier()

    def dump(agg_out, deg_out):
        @pl.when(s < NS - 1)
        def _():
            pltpu.sync_copy(agg_sh.at[rs_a], agg_out.at[rs_a])

        @pl.when(s == NS - 1)
        def _():
            pltpu.sync_copy(agg_sh.at[rs_b], agg_out.at[rs_b])

        if with_deg:
            @pl.when(s == 0)
            def _():
                pltpu.sync_copy(deg_sh, deg_out)

    @pl.when(c == 0)
    def _():
        dump(agg0_hbm, deg0_hbm if with_deg else None)

    @pl.when(c == 1)
    def _():
        dump(agg1_hbm, deg1_hbm if with_deg else None)


def _make_sc_agg(with_deg):
    f32 = jnp.float32
    outs = [jax.ShapeDtypeStruct((N, D), f32), jax.ShapeDtypeStruct((N, D), f32)]
    if with_deg:
        outs += [jax.ShapeDtypeStruct((DR, D), f32), jax.ShapeDtypeStruct((DR, D), f32)]
    scratch = [pltpu.VMEM_SHARED((N, D), f32)]
    if with_deg:
        scratch += [pltpu.VMEM_SHARED((DR, D), f32)]
    NB = NB1 if with_deg else NB2
    NI = 2 * NB
    scratch += [
        pltpu.VMEM((NI, 2, K), jnp.int32),
        pltpu.VMEM((NB, K, D), f32),
    ]
    if with_deg:
        scratch += [pltpu.VMEM((DR, D), f32), pltpu.VMEM((DR,), jnp.int32)]
    scratch += [pltpu.SemaphoreType.DMA((NI,)), pltpu.SemaphoreType.DMA((NB,)),
                pltpu.SemaphoreType.DMA((NB,))]

    mesh = plsc.VectorSubcoreMesh(core_axis_name="c", subcore_axis_name="s")
    return pl.kernel(
        functools.partial(_sc_agg_body, with_deg),
        out_type=tuple(outs),
        mesh=mesh,
        scratch_types=scratch,
    )


def _dense_body(agg0_ref, agg1_ref, deg0_ref, deg1_ref, h_ref,
                wl_ref, bl_ref, wr_ref, g_ref, beta_ref, out_ref):
    agg = agg0_ref[...] + agg1_ref[...]
    deg8 = deg0_ref[...] + deg1_ref[...]
    nrow = lax.broadcasted_iota(jnp.int32, (_BLK, _BLK // D), 0)
    rsel = lax.broadcasted_iota(jnp.int32, (_BLK, _BLK // D), 1)
    expand = (lax.shift_right_logical(nrow, 7) == rsel).astype(jnp.float32)
    rep = jnp.dot(expand, deg8, preferred_element_type=jnp.float32)
    ncol = lax.bitwise_and(
        lax.broadcasted_iota(jnp.int32, (_BLK, D), 0), D - 1)
    csel = lax.broadcasted_iota(jnp.int32, (_BLK, D), 1)
    deg = jnp.sum(jnp.where(ncol == csel, rep, 0.0), axis=1, keepdims=True)
    mean = agg / jnp.maximum(deg, 1.0)
    z = (jnp.dot(mean, wl_ref[...], preferred_element_type=jnp.float32)
         + jnp.dot(h_ref[...], wr_ref[...], preferred_element_type=jnp.float32)
         + bl_ref[...][None, :])
    mu = jnp.mean(z, axis=-1, keepdims=True)
    d = z - mu
    var = jnp.mean(d * d, axis=-1, keepdims=True)
    y = d * lax.rsqrt(var + EPS) * g_ref[...][None, :] + beta_ref[...][None, :]
    out_ref[...] = jnp.maximum(y, 0.0)


_BLK = 1024


def _dense(agg0, agg1, deg0, deg1, h, W_l, b_l, W_r, g, beta):
    grid = ((N + _BLK - 1) // _BLK,)
    fspec = pl.BlockSpec((_BLK, D), lambda i: (i, 0))
    dspec = pl.BlockSpec((_BLK // D, D), lambda i: (i, 0))
    wspec = pl.BlockSpec((D, D), lambda i: (0, 0))
    vspec = pl.BlockSpec((D,), lambda i: (0,))
    return pl.pallas_call(
        _dense_body,
        grid=grid,
        in_specs=[fspec, fspec, dspec, dspec, fspec, wspec, vspec, wspec, vspec, vspec],
        out_specs=fspec,
        out_shape=jax.ShapeDtypeStruct((N, D), jnp.float32),
    )(agg0, agg1, deg0, deg1, h, W_l, b_l, W_r, g, beta)


def kernel(x, edge_index, W0_l, b0_l, W0_r, g0, beta0, W1_l, b1_l, W1_r, g1, beta1):
    f32 = jnp.float32
    src = edge_index[0].astype(jnp.int32).reshape(NW, CHUNKS, K)
    dst = edge_index[1].astype(jnp.int32).reshape(NW, CHUNKS, K)
    z128 = jnp.zeros((RPS_LAST, D), f32)

    agg0, agg1, deg0, deg1 = _make_sc_agg(True)(x, src, dst, z128)
    h1 = _dense(agg0, agg1, deg0, deg1, x, W0_l, b0_l, W0_r, g0, beta0)
    agg0b, agg1b = _make_sc_agg(False)(h1, src, dst, z128)
    return _dense(agg0b, agg1b, deg0, deg1, h1, W1_l, b1_l, W1_r, g1, beta1)


# speedup vs baseline: 13.4107x; 1.0027x over previous
import functools

import jax
import jax.numpy as jnp
from jax import lax
from jax.experimental import pallas as pl
from jax.experimental.pallas import tpu as pltpu
from jax.experimental.pallas import tpu_sc as plsc

N = 10000
E = 320000
D = 128
EPS = 1e-5

NC = 2
NS = 16
NW = NC * NS
EPW = E // NW
K = 80
CHUNKS = EPW // K
NB1 = 3
NB2 = 4

RPS = 624
RPS_LAST = N - RPS * (NS - 1)
DR = 80
L = 16


def _sc_agg_body(with_deg, *refs):
    NB = NB1 if with_deg else NB2
    NI = 2 * NB
    if with_deg:
        (h_hbm, src_hbm, dst_hbm, z128_hbm,
         agg0_hbm, agg1_hbm, deg0_hbm, deg1_hbm,
         agg_sh, deg_sh, idx_v, rows_v, deg_v, ridx_v,
         isem, gsem, ssem) = refs
    else:
        (h_hbm, src_hbm, dst_hbm, z128_hbm,
         agg0_hbm, agg1_hbm,
         agg_sh, idx_v, rows_v, isem, gsem, ssem) = refs

    c = lax.axis_index("c")
    s = lax.axis_index("s")
    wid = s * NC + c

    def stage_idx(j, slot):
        base = pl.multiple_of(wid * EPW + j * K, 16)
        pltpu.async_copy(
            src_hbm.at[pl.ds(base, K)], idx_v.at[slot, 0], isem.at[slot])
        pltpu.async_copy(
            dst_hbm.at[pl.ds(base, K)], idx_v.at[slot, 1], isem.at[slot])

    def drain_idx(j, slot):
        base = pl.multiple_of(wid * EPW + j * K, 16)
        pltpu.make_async_copy(
            src_hbm.at[pl.ds(base, K)], idx_v.at[slot, 0], isem.at[slot]).wait()
        pltpu.make_async_copy(
            dst_hbm.at[pl.ds(base, K)], idx_v.at[slot, 1], isem.at[slot]).wait()

    for j in range(NI - 1):
        stage_idx(j, j)
    for b in range(NB - 1):
        drain_idx(b, b)
        pltpu.async_copy(h_hbm.at[idx_v.at[b, 0]], rows_v.at[b], gsem.at[b])

    rs_a = pl.ds(s * RPS, RPS)
    rs_b = pl.ds(RPS * (NS - 1), RPS_LAST)

    @pl.when(s < NS - 1)
    def _():
        pltpu.sync_copy(z128_hbm.at[pl.ds(0, RPS)], agg_sh.at[rs_a])

    @pl.when(s == NS - 1)
    def _():
        pltpu.sync_copy(z128_hbm, agg_sh.at[rs_b])

    if with_deg:
        zv = jnp.zeros((L,), jnp.float32)

        def zbody(i, carry):
            for j in range(D // L):
                deg_v[i, pl.ds(j * L, L)] = zv
            return carry

        lax.fori_loop(0, DR, zbody, 0)
        for j in range(DR // L):
            ridx_v[pl.ds(j * L, L)] = lax.iota(jnp.int32, L) + (j * L)

        @pl.when(s == 0)
        def _():
            pltpu.sync_copy(deg_v, deg_sh)

    plsc.subcore_barrier()

    lanes = lax.iota(jnp.int32, L)

    def deg_rmw(u):
        def sbody(g, carry2):
            idxv = idx_v[u, 1, pl.ds(g * L, L)]
            for jj in range(L):
                n = idxv[jj]
                r = lax.shift_right_logical(n, 7)
                off = lax.bitwise_and(lax.shift_right_logical(n, 4), 7) * L
                lane = lax.bitwise_and(n, 15)
                v = deg_v[r, pl.ds(off, L)]
                deg_v[r, pl.ds(off, L)] = v + jnp.where(
                    lanes == lane, 1.0, 0.0)
            return carry2

        lax.fori_loop(0, K // L, sbody, 0)

    def scatter_desc(u):
        return pltpu.make_async_copy(
            rows_v.at[u % NB], agg_sh.at[idx_v.at[u, 1]], ssem.at[u % NB])

    def body(g, carry):
        for u in range(NI):
            i = g * NI + u

            @pl.when(i < CHUNKS)
            def _():
                @pl.when(i >= 1)
                def _():
                    scatter_desc((u - 1) % NI).wait()

                @pl.when(i + NI - 1 < CHUNKS)
                def _():
                    stage_idx(i + NI - 1, (u - 1) % NI)

                @pl.when(i + NB - 1 < CHUNKS)
                def _():
                    drain_idx(i + NB - 1, (u + NB - 1) % NI)
                    pltpu.async_copy(
                        h_hbm.at[idx_v.at[(u + NB - 1) % NI, 0]],
                        rows_v.at[(u + NB - 1) % NB], gsem.at[(u + NB - 1) % NB])

                pltpu.make_async_copy(
                    h_hbm.at[idx_v.at[u, 0]], rows_v.at[u % NB],
                    gsem.at[u % NB]).wait()
                pltpu.async_copy(
                    rows_v.at[u % NB], agg_sh.at[idx_v.at[u, 1]],
                    ssem.at[u % NB], add=True)

                if with_deg:
                    deg_rmw(u)
        return carry

    lax.fori_loop(0, (CHUNKS + NI - 1) // NI, body, 0)
    scatter_desc((CHUNKS - 1) % NI).wait()

    if with_deg:
        pltpu.sync_copy(deg_v, deg_sh.at[ridx_v], add=True)
    plsc.subcore_barrier()

    def dump(agg_out, deg_out):
        @pl.when(s < NS - 1)
        def _():
            pltpu.sync_copy(agg_sh.at[rs_a], agg_out.at[rs_a])

        @pl.when(s == NS - 1)
        def _():
            pltpu.sync_copy(agg_sh.at[rs_b], agg_out.at[rs_b])

        if with_deg:
            @pl.when(s == 0)
            def _():
                pltpu.sync_copy(deg_sh, deg_out)

    @pl.when(c == 0)
    def _():
        dump(agg0_hbm, deg0_hbm if with_deg else None)

    @pl.when(c == 1)
    def _():
        dump(agg1_hbm, deg1_hbm if with_deg else None)


def _make_sc_agg(with_deg):
    f32 = jnp.float32
    outs = [jax.ShapeDtypeStruct((N, D), f32), jax.ShapeDtypeStruct((N, D), f32)]
    if with_deg:
        outs += [jax.ShapeDtypeStruct((DR, D), f32), jax.ShapeDtypeStruct((DR, D), f32)]
    scratch = [pltpu.VMEM_SHARED((N, D), f32)]
    if with_deg:
        scratch += [pltpu.VMEM_SHARED((DR, D), f32)]
    NB = NB1 if with_deg else NB2
    NI = 2 * NB
    scratch += [
        pltpu.VMEM((NI, 2, K), jnp.int32),
        pltpu.VMEM((NB, K, D), f32),
    ]
    if with_deg:
        scratch += [pltpu.VMEM((DR, D), f32), pltpu.VMEM((DR,), jnp.int32)]
    scratch += [pltpu.SemaphoreType.DMA((NI,)), pltpu.SemaphoreType.DMA((NB,)),
                pltpu.SemaphoreType.DMA((NB,))]

    mesh = plsc.VectorSubcoreMesh(core_axis_name="c", subcore_axis_name="s")
    return pl.kernel(
        functools.partial(_sc_agg_body, with_deg),
        out_type=tuple(outs),
        mesh=mesh,
        scratch_types=scratch,
    )


def _dense_body(agg0_ref, agg1_ref, deg0_ref, deg1_ref, h_ref,
                wl_ref, bl_ref, wr_ref, g_ref, beta_ref, out_ref):
    agg = agg0_ref[...] + agg1_ref[...]
    deg8 = deg0_ref[...] + deg1_ref[...]
    nrow = lax.broadcasted_iota(jnp.int32, (_BLK, _BLK // D), 0)
    rsel = lax.broadcasted_iota(jnp.int32, (_BLK, _BLK // D), 1)
    expand = (lax.shift_right_logical(nrow, 7) == rsel).astype(jnp.float32)
    rep = jnp.dot(expand, deg8, preferred_element_type=jnp.float32)
    ncol = lax.bitwise_and(
        lax.broadcasted_iota(jnp.int32, (_BLK, D), 0), D - 1)
    csel = lax.broadcasted_iota(jnp.int32, (_BLK, D), 1)
    deg = jnp.sum(jnp.where(ncol == csel, rep, 0.0), axis=1, keepdims=True)
    mean = agg / jnp.maximum(deg, 1.0)
    z = (jnp.dot(mean, wl_ref[...], preferred_element_type=jnp.float32)
         + jnp.dot(h_ref[...], wr_ref[...], preferred_element_type=jnp.float32)
         + bl_ref[...][None, :])
    mu = jnp.mean(z, axis=-1, keepdims=True)
    d = z - mu
    var = jnp.mean(d * d, axis=-1, keepdims=True)
    y = d * lax.rsqrt(var + EPS) * g_ref[...][None, :] + beta_ref[...][None, :]
    out_ref[...] = jnp.maximum(y, 0.0)


_BLK = 1024


def _dense(agg0, agg1, deg0, deg1, h, W_l, b_l, W_r, g, beta):
    grid = ((N + _BLK - 1) // _BLK,)
    fspec = pl.BlockSpec((_BLK, D), lambda i: (i, 0))
    dspec = pl.BlockSpec((_BLK // D, D), lambda i: (i, 0))
    wspec = pl.BlockSpec((D, D), lambda i: (0, 0))
    vspec = pl.BlockSpec((D,), lambda i: (0,))
    return pl.pallas_call(
        _dense_body,
        grid=grid,
        in_specs=[fspec, fspec, dspec, dspec, fspec, wspec, vspec, wspec, vspec, vspec],
        out_specs=fspec,
        out_shape=jax.ShapeDtypeStruct((N, D), jnp.float32),
    )(agg0, agg1, deg0, deg1, h, W_l, b_l, W_r, g, beta)


def kernel(x, edge_index, W0_l, b0_l, W0_r, g0, beta0, W1_l, b1_l, W1_r, g1, beta1):
    f32 = jnp.float32
    src = edge_index[0].astype(jnp.int32)
    dst = edge_index[1].astype(jnp.int32)
    z128 = jnp.zeros((RPS_LAST, D), f32)

    agg0, agg1, deg0, deg1 = _make_sc_agg(True)(x, src, dst, z128)
    h1 = _dense(agg0, agg1, deg0, deg1, x, W0_l, b0_l, W0_r, g0, beta0)
    agg0b, agg1b = _make_sc_agg(False)(h1, src, dst, z128)
    return _dense(agg0b, agg1b, deg0, deg1, h1, W1_l, b1_l, W1_r, g1, beta1)
